# baseline (speedup 1.0000x reference)
import functools

import jax
import jax.numpy as jnp
from jax import lax
from jax.experimental import pallas as pl
from jax.experimental.pallas import tpu as pltpu
from jax.experimental.pallas import tpu_sc as plsc

N = 4096
D = 256
T = 32768
OUT = 256
KSEL = 8
BMAX = 16

BLK = 512
BLKK = 4096
NI = N // BLK
NK = N // BLKK


def _prep_body(adj_ref, x_ref, w0_ref, xw_ref, adjb_ref, idx_ref, scale_ref,
               e_ref, bvec_ref, deg_scr):
    i = pl.program_id(0)
    adj = adj_ref[...]
    deg_scr[pl.ds(i * (BLK // 128), BLK // 128), :] = (
        jnp.sum(adj, axis=1).reshape(BLK // 128, 128))
    adjb_ref[...] = adj.astype(jnp.bfloat16)
    xw_ref[...] = jnp.dot(x_ref[...], w0_ref[...],
                          preferred_element_type=jnp.float32
                          ).astype(jnp.bfloat16)

    @pl.when(i == NI - 1)
    def _():
        d = deg_scr[...]
        gid = (lax.broadcasted_iota(jnp.int32, d.shape, 0) * 128
               + lax.broadcasted_iota(jnp.int32, d.shape, 1))
        cur = d
        vals = []
        for j in range(BMAX):
            m = jnp.max(cur)
            ix = jnp.min(jnp.where(cur == m, gid, jnp.int32(2**30)))
            vals.append(m)
            idx_ref[j] = ix
            cur = jnp.where(gid == ix, -jnp.inf, cur)
        b_count = jnp.int32(KSEL)
        for j in range(KSEL, BMAX):
            b_count = b_count + (vals[j] == vals[KSEL - 1]).astype(jnp.int32)
        inv_sqrt_b = lax.rsqrt(b_count.astype(jnp.float32))
        for b in range(BMAX):
            scale_ref[b] = jnp.where(b < b_count, inv_sqrt_b, 0.0)
        bvec_ref[...] = jnp.zeros((BMAX,), jnp.int32) + b_count
        colid = lax.broadcasted_iota(jnp.int32, (1, BMAX), 1)
        idxvec = jnp.zeros((1, BMAX), jnp.int32)
        for b in range(BMAX):
            idxvec = jnp.where(colid == b, idx_ref[b], idxvec)
        rowid = lax.broadcasted_iota(jnp.int32, (N, BMAX), 0)
        e_ref[...] = (rowid == idxvec).astype(jnp.bfloat16)


def _prep(adj_t, X, W0):
    return pl.pallas_call(
        _prep_body,
        grid=(NI,),
        in_specs=[
            pl.BlockSpec((BLK, N), lambda i: (i, 0)),
            pl.BlockSpec((BLK, D), lambda i: (i, 0)),
            pl.BlockSpec((D, OUT), lambda i: (0, 0)),
        ],
        out_specs=[
            pl.BlockSpec((BLK, OUT), lambda i: (i, 0)),
            pl.BlockSpec((BLK, N), lambda i: (i, 0)),
            pl.BlockSpec(memory_space=pltpu.SMEM),
            pl.BlockSpec(memory_space=pltpu.SMEM),
            pl.BlockSpec((N, BMAX), lambda i: (0, 0)),
            pl.BlockSpec((BMAX,), lambda i: (0,)),
        ],
        out_shape=[
            jax.ShapeDtypeStruct((N, OUT), jnp.bfloat16),
            jax.ShapeDtypeStruct((N, N), jnp.bfloat16),
            jax.ShapeDtypeStruct((BMAX,), jnp.int32),
            jax.ShapeDtypeStruct((BMAX,), jnp.float32),
            jax.ShapeDtypeStruct((N, BMAX), jnp.bfloat16),
            jax.ShapeDtypeStruct((BMAX,), jnp.int32),
        ],
        scratch_shapes=[pltpu.VMEM((32, 128), jnp.float32)],
    )(adj_t, X, W0)


def _pack_words(x):
    lo = lax.bitcast_convert_type(x[:, :128].astype(jnp.bfloat16),
                                  jnp.uint16).astype(jnp.uint32)
    hi = lax.bitcast_convert_type(x[:, 128:].astype(jnp.bfloat16),
                                  jnp.uint16).astype(jnp.uint32)
    return lax.bitcast_convert_type(lo | (hi << 16), jnp.int32)


def _table_body(scale_ref, adj_ref, xw_ref, e_ref, wrow_ref, lns_ref,
                lnb_ref, table_ref, acc_ref, accg_ref):
    k = pl.program_id(1)

    @pl.when(k == 0)
    def _():
        acc_ref[...] = jnp.zeros_like(acc_ref)
        accg_ref[...] = jnp.zeros_like(accg_ref)

    adj = adj_ref[...]
    acc_ref[...] += jnp.dot(adj, xw_ref[...],
                            preferred_element_type=jnp.float32)
    accg_ref[...] += jnp.dot(adj, e_ref[...],
                             preferred_element_type=jnp.float32)

    @pl.when(k == NK - 1)
    def _():
        a = acc_ref[...]
        g = accg_ref[...]
        w = wrow_ref[...]
        lns = lns_ref[...]
        lnb = lnb_ref[...]
        def emit(b):
            sb = scale_ref[b]
            h = jnp.maximum(a + g[:, b:b + 1] * w, 0.0)
            mu = jnp.mean(h, axis=1, keepdims=True)
            msq = jnp.mean(h * h, axis=1, keepdims=True)
            c1 = lax.rsqrt(msq - mu * mu + 1e-5) * sb
            table_ref[:, b, :] = _pack_words(
                (h - mu) * c1 * lns + lnb * sb)

        for b in range(KSEL):
            emit(b)
        for b in range(KSEL, BMAX):
            live = scale_ref[b] != 0.0

            @pl.when(live)
            def _(b=b):
                emit(b)

            @pl.when(jnp.logical_not(live))
            def _(b=b):
                table_ref[:, b, :] = jnp.zeros((BLK, 128), jnp.int32)


def _table(scales, adj_bf, xw, e, wrow, lns, lnb):
    return pl.pallas_call(
        _table_body,
        grid=(NI, NK),
        in_specs=[
            pl.BlockSpec(memory_space=pltpu.SMEM),
            pl.BlockSpec((BLK, BLKK), lambda i, k: (i, k)),
            pl.BlockSpec((BLKK, OUT), lambda i, k: (k, 0)),
            pl.BlockSpec((BLKK, BMAX), lambda i, k: (k, 0)),
            pl.BlockSpec((1, OUT), lambda i, k: (0, 0)),
            pl.BlockSpec((1, OUT), lambda i, k: (0, 0)),
            pl.BlockSpec((1, OUT), lambda i, k: (0, 0)),
        ],
        out_specs=pl.BlockSpec((BLK, BMAX, 128), lambda i, k: (i, 0, 0)),
        out_shape=jax.ShapeDtypeStruct((N, BMAX, 128), jnp.int32),
        scratch_shapes=[
            pltpu.VMEM((BLK, OUT), jnp.float32),
            pltpu.VMEM((BLK, BMAX), jnp.float32),
        ],
    )(scales, adj_bf, xw, e, wrow, lns, lnb)


NW = 32
TPW = T // NW
CH = 8
NCH = TPW // CH
NBUF = 4
BH = BMAX // 2


def _sc_body(table_hbm, tups_hbm, tupd_hbm, bvec_hbm, out_hbm, idx_s, idx_d,
             idx_s1, idx_d1, sbuf0, sbuf1, sbuf2, sbuf3, dbuf0, dbuf1, dbuf2,
             dbuf3, orows, obuf, bc_v, sem_s0, sem_s1, sem_s2, sem_s3,
             sem_d0, sem_d1, sem_d2, sem_d3):
    wid = lax.axis_index("s") * 2 + lax.axis_index("c")
    base = wid * TPW
    pltpu.sync_copy(tups_hbm.at[pl.ds(base, TPW)], idx_s)
    pltpu.sync_copy(tupd_hbm.at[pl.ds(base, TPW)], idx_d)
    pltpu.sync_copy(bvec_hbm, bc_v)
    bcnt = jnp.max(bc_v[...])

    def dbl(j, carry):
        sl = pl.ds(j * 16, 16)
        vs = idx_s[sl]
        vd = idx_d[sl]
        idx_s[sl] = vs + vs
        idx_d[sl] = vd + vd
        idx_s1[sl] = vs + vs + 1
        idx_d1[sl] = vd + vd + 1
        return carry

    lax.fori_loop(0, TPW // 16, dbl, 0)

    sbufs = (sbuf0, sbuf1, sbuf2, sbuf3)
    dbufs = (dbuf0, dbuf1, dbuf2, dbuf3)
    sems_s = (sem_s0, sem_s1, sem_s2, sem_s3)
    sems_d = (sem_d0, sem_d1, sem_d2, sem_d3)

    def run_pass(iss, isd, accumulate):
        def fire(c, p):
            co = jnp.minimum(c, NCH - 1) * CH
            pltpu.async_copy(table_hbm.at[iss.at[pl.ds(co, CH)]],
                             sbufs[p], sems_s[p])
            pltpu.async_copy(table_hbm.at[isd.at[pl.ds(co, CH)]],
                             dbufs[p], sems_d[p])

        def wait(c, p):
            co = jnp.minimum(c, NCH - 1) * CH
            pltpu.make_async_copy(table_hbm.at[iss.at[pl.ds(co, CH)]],
                                  sbufs[p], sems_s[p]).wait()
            pltpu.make_async_copy(table_hbm.at[isd.at[pl.ds(co, CH)]],
                                  dbufs[p], sems_d[p]).wait()

        def compute(p, co):
            buf_s = sbufs[p]
            buf_d = dbufs[p]
            if accumulate:
                pltpu.sync_copy(out_hbm.at[pl.ds(base + co, CH)], obuf)

            def tup(t, carry):
                for w in range(8):
                    lo = w * 16
                    if accumulate:
                        acc_e = obuf[t, pl.ds(lo, 16)]
                        acc_o = obuf[t, pl.ds(128 + lo, 16)]
                    else:
                        acc_e = jnp.zeros((16,), jnp.float32)
                        acc_o = jnp.zeros((16,), jnp.float32)
                    for b in range(BH):
                        sv = plsc.bitcast(buf_s[t, b, pl.ds(lo, 16)],
                                          jnp.bfloat16)
                        dv = plsc.bitcast(buf_d[t, b, pl.ds(lo, 16)],
                                          jnp.bfloat16)
                        pe, po = plsc.unpack(
                            sv * dv, format=plsc.PackFormat.INTERLEAVED)
                        acc_e = acc_e + pe
                        acc_o = acc_o + po
                    orows[t, pl.ds(lo, 16)] = acc_e
                    orows[t, pl.ds(128 + lo, 16)] = acc_o
                return carry

            lax.fori_loop(0, CH, tup, 0)
            pltpu.sync_copy(orows, out_hbm.at[pl.ds(base + co, CH)])

        for p in range(NBUF - 1):
            fire(p, p)

        def ring(cg, carry):
            c0 = cg * NBUF
            for p in range(NBUF):
                fire(c0 + p + NBUF - 1, (p + NBUF - 1) % NBUF)
                wait(c0 + p, p)
                compute(p, (c0 + p) * CH)
            return carry

        lax.fori_loop(0, NCH // NBUF, ring, 0)
        for p in range(NBUF - 1):
            wait(NCH, p)

    run_pass(idx_s, idx_d, False)

    @pl.when(bcnt > KSEL)
    def _():
        run_pass(idx_s1, idx_d1, True)


@functools.cache
def _sc_gather():
    return pl.kernel(
        _sc_body,
        out_type=jax.ShapeDtypeStruct((T, OUT), jnp.float32),
        mesh=plsc.VectorSubcoreMesh(core_axis_name="c", subcore_axis_name="s"),
        compiler_params=pltpu.CompilerParams(needs_layout_passes=False),
        scratch_types=[
            pltpu.VMEM((TPW,), jnp.int32),
            pltpu.VMEM((TPW,), jnp.int32),
            pltpu.VMEM((TPW,), jnp.int32),
            pltpu.VMEM((TPW,), jnp.int32),
            pltpu.VMEM((CH, BH, 128), jnp.int32),
            pltpu.VMEM((CH, BH, 128), jnp.int32),
            pltpu.VMEM((CH, BH, 128), jnp.int32),
            pltpu.VMEM((CH, BH, 128), jnp.int32),
            pltpu.VMEM((CH, BH, 128), jnp.int32),
            pltpu.VMEM((CH, BH, 128), jnp.int32),
            pltpu.VMEM((CH, BH, 128), jnp.int32),
            pltpu.VMEM((CH, BH, 128), jnp.int32),
            pltpu.VMEM((CH, OUT), jnp.float32),
            pltpu.VMEM((CH, OUT), jnp.float32),
            pltpu.VMEM((BMAX,), jnp.int32),
            pltpu.SemaphoreType.DMA,
            pltpu.SemaphoreType.DMA,
            pltpu.SemaphoreType.DMA,
            pltpu.SemaphoreType.DMA,
            pltpu.SemaphoreType.DMA,
            pltpu.SemaphoreType.DMA,
            pltpu.SemaphoreType.DMA,
            pltpu.SemaphoreType.DMA,
        ],
    )


def kernel(X, adj_t, tuples_coo, W, ln_scale, ln_bias):
    W0 = W[:D]
    wrow = W[D:D + 1]
    xw, adj_bf, idx, scales, e, bvec = _prep(adj_t, X, W0)
    del idx
    table = _table(scales, adj_bf, xw, e, wrow,
                   ln_scale.reshape(1, OUT), ln_bias.reshape(1, OUT))
    tup = tuples_coo.astype(jnp.int32)
    return _sc_gather()(table.reshape(2 * N, BH, 128), tup[0], tup[1], bvec)


# speedup vs baseline: 1.0200x; 1.0200x over previous
import functools

import jax
import jax.numpy as jnp
from jax import lax
from jax.experimental import pallas as pl
from jax.experimental.pallas import tpu as pltpu
from jax.experimental.pallas import tpu_sc as plsc

N = 4096
D = 256
T = 32768
OUT = 256
KSEL = 8
BMAX = 16

BLK = 512
BLKK = 4096
NI = N // BLK
NK = N // BLKK


def _prep_body(adj_ref, x_ref, w0_ref, xw_ref, adjb_ref, idx_ref, scale_ref,
               e_ref, bvec_ref, deg_scr):
    i = pl.program_id(0)
    adj = adj_ref[...]
    deg_scr[pl.ds(i * (BLK // 128), BLK // 128), :] = (
        jnp.sum(adj, axis=1).reshape(BLK // 128, 128))
    adjb_ref[...] = adj.astype(jnp.bfloat16)
    xw_ref[...] = jnp.dot(x_ref[...], w0_ref[...],
                          preferred_element_type=jnp.float32
                          ).astype(jnp.bfloat16)

    @pl.when(i == NI - 1)
    def _():
        d = deg_scr[...]
        gid = (lax.broadcasted_iota(jnp.int32, d.shape, 0) * 128
               + lax.broadcasted_iota(jnp.int32, d.shape, 1))
        cur = d
        vals = []
        for j in range(BMAX):
            m = jnp.max(cur)
            ix = jnp.min(jnp.where(cur == m, gid, jnp.int32(2**30)))
            vals.append(m)
            idx_ref[j] = ix
            cur = jnp.where(gid == ix, -jnp.inf, cur)
        b_count = jnp.int32(KSEL)
        for j in range(KSEL, BMAX):
            b_count = b_count + (vals[j] == vals[KSEL - 1]).astype(jnp.int32)
        inv_sqrt_b = lax.rsqrt(b_count.astype(jnp.float32))
        for b in range(BMAX):
            scale_ref[b] = jnp.where(b < b_count, inv_sqrt_b, 0.0)
        bvec_ref[...] = jnp.zeros((BMAX,), jnp.int32) + b_count
        colid = lax.broadcasted_iota(jnp.int32, (1, BMAX), 1)
        idxvec = jnp.zeros((1, BMAX), jnp.int32)
        for b in range(BMAX):
            idxvec = jnp.where(colid == b, idx_ref[b], idxvec)
        rowid = lax.broadcasted_iota(jnp.int32, (N, BMAX), 0)
        e_ref[...] = (rowid == idxvec).astype(jnp.bfloat16)


def _prep(adj_t, X, W0):
    return pl.pallas_call(
        _prep_body,
        grid=(NI,),
        in_specs=[
            pl.BlockSpec((BLK, N), lambda i: (i, 0)),
            pl.BlockSpec((BLK, D), lambda i: (i, 0)),
            pl.BlockSpec((D, OUT), lambda i: (0, 0)),
        ],
        out_specs=[
            pl.BlockSpec((BLK, OUT), lambda i: (i, 0)),
            pl.BlockSpec((BLK, N), lambda i: (i, 0)),
            pl.BlockSpec(memory_space=pltpu.SMEM),
            pl.BlockSpec(memory_space=pltpu.SMEM),
            pl.BlockSpec((N, BMAX), lambda i: (0, 0)),
            pl.BlockSpec((BMAX,), lambda i: (0,)),
        ],
        out_shape=[
            jax.ShapeDtypeStruct((N, OUT), jnp.bfloat16),
            jax.ShapeDtypeStruct((N, N), jnp.bfloat16),
            jax.ShapeDtypeStruct((BMAX,), jnp.int32),
            jax.ShapeDtypeStruct((BMAX,), jnp.float32),
            jax.ShapeDtypeStruct((N, BMAX), jnp.bfloat16),
            jax.ShapeDtypeStruct((BMAX,), jnp.int32),
        ],
        scratch_shapes=[pltpu.VMEM((32, 128), jnp.float32)],
    )(adj_t, X, W0)


def _pack_words(x):
    lo = lax.bitcast_convert_type(x[:, :128].astype(jnp.bfloat16),
                                  jnp.uint16).astype(jnp.uint32)
    hi = lax.bitcast_convert_type(x[:, 128:].astype(jnp.bfloat16),
                                  jnp.uint16).astype(jnp.uint32)
    return lax.bitcast_convert_type(lo | (hi << 16), jnp.int32)


def _table_body(scale_ref, adj_ref, xw_ref, e_ref, wrow_ref, lns_ref,
                lnb_ref, table_ref, acc_ref, accg_ref):
    k = pl.program_id(1)

    @pl.when(k == 0)
    def _():
        acc_ref[...] = jnp.zeros_like(acc_ref)
        accg_ref[...] = jnp.zeros_like(accg_ref)

    adj = adj_ref[...]
    acc_ref[...] += jnp.dot(adj, xw_ref[...],
                            preferred_element_type=jnp.float32)
    accg_ref[...] += jnp.dot(adj, e_ref[...],
                             preferred_element_type=jnp.float32)

    @pl.when(k == NK - 1)
    def _():
        a = acc_ref[...]
        g = accg_ref[...]
        w = wrow_ref[...]
        lns = lns_ref[...]
        lnb = lnb_ref[...]
        def emit(b):
            sb = scale_ref[b]
            h = jnp.maximum(a + g[:, b:b + 1] * w, 0.0)
            mu = jnp.mean(h, axis=1, keepdims=True)
            msq = jnp.mean(h * h, axis=1, keepdims=True)
            c1 = lax.rsqrt(msq - mu * mu + 1e-5) * sb
            table_ref[:, b, :] = _pack_words(
                (h - mu) * c1 * lns + lnb * sb)

        for b in range(KSEL):
            emit(b)
        for b in range(KSEL, BMAX):
            live = scale_ref[b] != 0.0

            @pl.when(live)
            def _(b=b):
                emit(b)

            @pl.when(jnp.logical_not(live))
            def _(b=b):
                table_ref[:, b, :] = jnp.zeros((BLK, 128), jnp.int32)


def _table(scales, adj_bf, xw, e, wrow, lns, lnb):
    return pl.pallas_call(
        _table_body,
        grid=(NI, NK),
        in_specs=[
            pl.BlockSpec(memory_space=pltpu.SMEM),
            pl.BlockSpec((BLK, BLKK), lambda i, k: (i, k)),
            pl.BlockSpec((BLKK, OUT), lambda i, k: (k, 0)),
            pl.BlockSpec((BLKK, BMAX), lambda i, k: (k, 0)),
            pl.BlockSpec((1, OUT), lambda i, k: (0, 0)),
            pl.BlockSpec((1, OUT), lambda i, k: (0, 0)),
            pl.BlockSpec((1, OUT), lambda i, k: (0, 0)),
        ],
        out_specs=pl.BlockSpec((BLK, BMAX, 128), lambda i, k: (i, 0, 0)),
        out_shape=jax.ShapeDtypeStruct((N, BMAX, 128), jnp.int32),
        scratch_shapes=[
            pltpu.VMEM((BLK, OUT), jnp.float32),
            pltpu.VMEM((BLK, BMAX), jnp.float32),
        ],
    )(scales, adj_bf, xw, e, wrow, lns, lnb)


NW = 32
TPW = T // NW
CH = 16
NCH = TPW // CH
NBUF = 2
BH = BMAX // 2


def _sc_body(table_hbm, tups_hbm, tupd_hbm, bvec_hbm, out_hbm, idx_s, idx_d,
             idx_s1, idx_d1, sbuf0, sbuf1, dbuf0, dbuf1, orows, obuf, bc_v,
             sem_s0, sem_s1, sem_d0, sem_d1):
    wid = lax.axis_index("s") * 2 + lax.axis_index("c")
    base = wid * TPW
    pltpu.sync_copy(tups_hbm.at[pl.ds(base, TPW)], idx_s)
    pltpu.sync_copy(tupd_hbm.at[pl.ds(base, TPW)], idx_d)
    pltpu.sync_copy(bvec_hbm, bc_v)
    bcnt = jnp.max(bc_v[...])

    def dbl(j, carry):
        sl = pl.ds(j * 16, 16)
        vs = idx_s[sl]
        vd = idx_d[sl]
        idx_s[sl] = vs + vs
        idx_d[sl] = vd + vd
        idx_s1[sl] = vs + vs + 1
        idx_d1[sl] = vd + vd + 1
        return carry

    lax.fori_loop(0, TPW // 16, dbl, 0)

    sbufs = (sbuf0, sbuf1)
    dbufs = (dbuf0, dbuf1)
    sems_s = (sem_s0, sem_s1)
    sems_d = (sem_d0, sem_d1)

    def run_pass(iss, isd, accumulate):
        def fire(c, p):
            co = jnp.minimum(c, NCH - 1) * CH
            pltpu.async_copy(table_hbm.at[iss.at[pl.ds(co, CH)]],
                             sbufs[p], sems_s[p])
            pltpu.async_copy(table_hbm.at[isd.at[pl.ds(co, CH)]],
                             dbufs[p], sems_d[p])

        def wait(c, p):
            co = jnp.minimum(c, NCH - 1) * CH
            pltpu.make_async_copy(table_hbm.at[iss.at[pl.ds(co, CH)]],
                                  sbufs[p], sems_s[p]).wait()
            pltpu.make_async_copy(table_hbm.at[isd.at[pl.ds(co, CH)]],
                                  dbufs[p], sems_d[p]).wait()

        def compute(p, co):
            buf_s = sbufs[p]
            buf_d = dbufs[p]
            if accumulate:
                pltpu.sync_copy(out_hbm.at[pl.ds(base + co, CH)], obuf)

            def tup(t, carry):
                for w in range(8):
                    lo = w * 16
                    if accumulate:
                        acc_e = obuf[t, pl.ds(lo, 16)]
                        acc_o = obuf[t, pl.ds(128 + lo, 16)]
                    else:
                        acc_e = jnp.zeros((16,), jnp.float32)
                        acc_o = jnp.zeros((16,), jnp.float32)
                    for b in range(BH):
                        sv = plsc.bitcast(buf_s[t, b, pl.ds(lo, 16)],
                                          jnp.bfloat16)
                        dv = plsc.bitcast(buf_d[t, b, pl.ds(lo, 16)],
                                          jnp.bfloat16)
                        pe, po = plsc.unpack(
                            sv * dv, format=plsc.PackFormat.INTERLEAVED)
                        acc_e = acc_e + pe
                        acc_o = acc_o + po
                    orows[t, pl.ds(lo, 16)] = acc_e
                    orows[t, pl.ds(128 + lo, 16)] = acc_o
                return carry

            lax.fori_loop(0, CH, tup, 0)
            pltpu.sync_copy(orows, out_hbm.at[pl.ds(base + co, CH)])

        for p in range(NBUF - 1):
            fire(p, p)

        def ring(cg, carry):
            c0 = cg * NBUF
            for p in range(NBUF):
                fire(c0 + p + NBUF - 1, (p + NBUF - 1) % NBUF)
                wait(c0 + p, p)
                compute(p, (c0 + p) * CH)
            return carry

        lax.fori_loop(0, NCH // NBUF, ring, 0)
        for p in range(NBUF - 1):
            wait(NCH, p)

    run_pass(idx_s, idx_d, False)

    @pl.when(bcnt > KSEL)
    def _():
        run_pass(idx_s1, idx_d1, True)


@functools.cache
def _sc_gather():
    return pl.kernel(
        _sc_body,
        out_type=jax.ShapeDtypeStruct((T, OUT), jnp.float32),
        mesh=plsc.VectorSubcoreMesh(core_axis_name="c", subcore_axis_name="s"),
        compiler_params=pltpu.CompilerParams(needs_layout_passes=False),
        scratch_types=[
            pltpu.VMEM((TPW,), jnp.int32),
            pltpu.VMEM((TPW,), jnp.int32),
            pltpu.VMEM((TPW,), jnp.int32),
            pltpu.VMEM((TPW,), jnp.int32),
            pltpu.VMEM((CH, BH, 128), jnp.int32),
            pltpu.VMEM((CH, BH, 128), jnp.int32),
            pltpu.VMEM((CH, BH, 128), jnp.int32),
            pltpu.VMEM((CH, BH, 128), jnp.int32),
            pltpu.VMEM((CH, OUT), jnp.float32),
            pltpu.VMEM((CH, OUT), jnp.float32),
            pltpu.VMEM((BMAX,), jnp.int32),
            pltpu.SemaphoreType.DMA,
            pltpu.SemaphoreType.DMA,
            pltpu.SemaphoreType.DMA,
            pltpu.SemaphoreType.DMA,
        ],
    )


def kernel(X, adj_t, tuples_coo, W, ln_scale, ln_bias):
    W0 = W[:D]
    wrow = W[D:D + 1]
    xw, adj_bf, idx, scales, e, bvec = _prep(adj_t, X, W0)
    del idx
    table = _table(scales, adj_bf, xw, e, wrow,
                   ln_scale.reshape(1, OUT), ln_bias.reshape(1, OUT))
    tup = tuples_coo.astype(jnp.int32)
    return _sc_gather()(table.reshape(2 * N, BH, 128), tup[0], tup[1], bvec)
